# Initial kernel scaffold; baseline (speedup 1.0000x reference)
import jax
import jax.numpy as jnp
from jax.experimental import pallas as pl


def kernel(x, edge_index, rel_type, W_fc, W_self, attn_w):
    raise NotImplementedError("write your pallas kernel here")


# speedup vs baseline: 3.0209x; 3.0209x over previous
import functools

import jax
import jax.numpy as jnp
from jax import lax
from jax.experimental import pallas as pl
from jax.experimental.pallas import tpu as pltpu
from jax.experimental.pallas import tpu_sc as plsc

N = 10000
E = 160000
IN_DIM = 128
OUT_DIM = 64
HEADS = 5
NUM_RELS = 20
HP = 16
FEAT = HEADS * OUT_DIM
HALF = FEAT // 2
NSC = 2
NTILE = 16
EPT = E // NTILE
C = 80
NCHUNK = EPT // C
ROWS_PT = N // NTILE
ROWS_LAST0 = (NTILE - 1) * (ROWS_PT - 1)
ROWS_LAST = N - ROWS_LAST0
JBLK = HALF // 16

ROW_BLK = 1000


def _tc_body(x_ref, wfc_ref, wu_ref, wv_ref, wself_ref,
             z_ref, u_ref, v_ref, s_ref):
    x = x_ref[...]
    z = jnp.dot(x, wfc_ref[...], preferred_element_type=jnp.float32)
    z_ref[...] = z
    u_ref[...] = jnp.dot(z, wu_ref[...], preferred_element_type=jnp.float32)
    v_ref[...] = jnp.dot(z, wv_ref[...], preferred_element_type=jnp.float32)
    s_ref[...] = jnp.dot(x, wself_ref[...], preferred_element_type=jnp.float32)


def _dense_stage(x, wfc_t, wu_mat, wv_mat, wself_t):
    grid = (N // ROW_BLK,)
    return pl.pallas_call(
        _tc_body,
        grid=grid,
        in_specs=[
            pl.BlockSpec((ROW_BLK, IN_DIM), lambda i: (i, 0)),
            pl.BlockSpec((IN_DIM, OUT_DIM), lambda i: (0, 0)),
            pl.BlockSpec((OUT_DIM, NUM_RELS * HP), lambda i: (0, 0)),
            pl.BlockSpec((OUT_DIM, NUM_RELS * HP), lambda i: (0, 0)),
            pl.BlockSpec((IN_DIM, FEAT), lambda i: (0, 0)),
        ],
        out_specs=[
            pl.BlockSpec((ROW_BLK, OUT_DIM), lambda i: (i, 0)),
            pl.BlockSpec((ROW_BLK, NUM_RELS * HP), lambda i: (i, 0)),
            pl.BlockSpec((ROW_BLK, NUM_RELS * HP), lambda i: (i, 0)),
            pl.BlockSpec((ROW_BLK, FEAT), lambda i: (i, 0)),
        ],
        out_shape=[
            jax.ShapeDtypeStruct((N, OUT_DIM), jnp.float32),
            jax.ShapeDtypeStruct((N, NUM_RELS * HP), jnp.float32),
            jax.ShapeDtypeStruct((N, NUM_RELS * HP), jnp.float32),
            jax.ShapeDtypeStruct((N, FEAT), jnp.float32),
        ],
    )(x, wfc_t, wu_mat, wv_mat, wself_t)


def _sc_edge_body(z_hbm, u_hbm, v_hbm, s0_hbm, s1_hbm, src_hbm, dst_hbm,
                  rel_hbm, out0_hbm, out1_hbm, acc, src_v, dst_v, rel_v,
                  uidx_v, vidx_v, u_v, v_v, zs_v, msg_v, sem):
    c = lax.axis_index("c")
    t = lax.axis_index("s")
    col0 = c * HALF

    heads = [(col0 + 16 * j) // OUT_DIM for j in range(JBLK)]
    doffs = [(col0 + 16 * j) % OUT_DIM for j in range(JBLK)]
    head_idxvecs = [jnp.full((16,), h, dtype=jnp.int32) for h in heads]

    r0 = t * (ROWS_PT - 1)

    def _strip_copy(src_at, dst_at):
        @pl.when(t < NTILE - 1)
        def _small():
            pltpu.sync_copy(src_at(r0, ROWS_PT - 1), dst_at(r0, ROWS_PT - 1))

        @pl.when(t == NTILE - 1)
        def _last():
            pltpu.sync_copy(src_at(ROWS_LAST0, ROWS_LAST),
                            dst_at(ROWS_LAST0, ROWS_LAST))

    @pl.when(c == 0)
    def _init0():
        _strip_copy(lambda r, n: s0_hbm.at[pl.ds(r, n)],
                    lambda r, n: acc.at[pl.ds(r, n)])

    @pl.when(c == 1)
    def _init1():
        _strip_copy(lambda r, n: s1_hbm.at[pl.ds(r, n)],
                    lambda r, n: acc.at[pl.ds(r, n)])

    plsc.subcore_barrier()

    base = t * EPT

    def chunk_body(k, carry):
        off = base + k * C
        pltpu.sync_copy(src_hbm.at[pl.ds(off, C)], src_v)
        pltpu.sync_copy(dst_hbm.at[pl.ds(off, C)], dst_v)
        pltpu.sync_copy(rel_hbm.at[pl.ds(off, C)], rel_v)
        for i in range(C // 16):
            sl = pl.ds(i * 16, 16)
            r16 = rel_v[sl]
            uidx_v[sl] = src_v[sl] * NUM_RELS + r16
            vidx_v[sl] = dst_v[sl] * NUM_RELS + r16
        cp1 = pltpu.async_copy(u_hbm.at[uidx_v], u_v, sem)
        cp2 = pltpu.async_copy(v_hbm.at[vidx_v], v_v, sem)
        cp3 = pltpu.async_copy(z_hbm.at[src_v], zs_v, sem)
        cp1.wait()
        cp2.wait()
        cp3.wait()

        def edge_body(e, carry2):
            a = u_v[e, :] + v_v[e, :]
            att = jnp.maximum(a, 0.01 * a)
            for j in range(JBLK):
                splat = att.at[head_idxvecs[j]].get(mode="promise_in_bounds")
                z16 = zs_v[e, pl.ds(doffs[j], 16)]
                msg_v[e, pl.ds(j * 16, 16)] = splat * z16
            return carry2

        lax.fori_loop(0, C, edge_body, 0)
        pltpu.sync_copy(msg_v, acc.at[dst_v], add=True)
        return carry

    lax.fori_loop(0, NCHUNK, chunk_body, 0)
    plsc.subcore_barrier()

    @pl.when(c == 0)
    def _out0():
        _strip_copy(lambda r, n: acc.at[pl.ds(r, n)],
                    lambda r, n: out0_hbm.at[pl.ds(r, n)])

    @pl.when(c == 1)
    def _out1():
        _strip_copy(lambda r, n: acc.at[pl.ds(r, n)],
                    lambda r, n: out1_hbm.at[pl.ds(r, n)])


def _build_sc_kernel():
    return pl.kernel(
        _sc_edge_body,
        out_type=[jax.ShapeDtypeStruct((N, HALF), jnp.float32),
                  jax.ShapeDtypeStruct((N, HALF), jnp.float32)],
        mesh=plsc.VectorSubcoreMesh(core_axis_name="c", subcore_axis_name="s",
                                    num_cores=NSC, num_subcores=NTILE),
        scratch_types=[
            pltpu.VMEM_SHARED((N, HALF), jnp.float32),
            pltpu.VMEM((C,), jnp.int32),
            pltpu.VMEM((C,), jnp.int32),
            pltpu.VMEM((C,), jnp.int32),
            pltpu.VMEM((C,), jnp.int32),
            pltpu.VMEM((C,), jnp.int32),
            pltpu.VMEM((C, HP), jnp.float32),
            pltpu.VMEM((C, HP), jnp.float32),
            pltpu.VMEM((C, OUT_DIM), jnp.float32),
            pltpu.VMEM((C, HALF), jnp.float32),
            pltpu.SemaphoreType.DMA,
        ],
        compiler_params=pltpu.CompilerParams(use_tc_tiling_on_sc=False),
    )


@jax.jit
def kernel(x, edge_index, rel_type, W_fc, W_self, attn_w):
    wfc_t = W_fc.T
    pad = ((0, 0), (0, 0), (0, HP - HEADS))
    wu_mat = jnp.pad(attn_w[:, :OUT_DIM, :], pad)
    wu_mat = wu_mat.transpose(1, 0, 2).reshape(OUT_DIM, NUM_RELS * HP)
    wv_mat = jnp.pad(attn_w[:, OUT_DIM:, :], pad)
    wv_mat = wv_mat.transpose(1, 0, 2).reshape(OUT_DIM, NUM_RELS * HP)
    wself_t = W_self.T

    z, u_tab, v_tab, s_tab = _dense_stage(x, wfc_t, wu_mat, wv_mat, wself_t)
    u_tab = u_tab.reshape(N * NUM_RELS, HP)
    v_tab = v_tab.reshape(N * NUM_RELS, HP)

    src = edge_index[0]
    dst = edge_index[1]
    s0 = s_tab[:, :HALF]
    s1 = s_tab[:, HALF:]
    out0, out1 = _build_sc_kernel()(z, u_tab, v_tab, s0, s1, src, dst,
                                    rel_type)
    return jnp.concatenate([out0, out1], axis=1)
